# baseline (speedup 1.0000x reference)
import functools

import jax
import jax.numpy as jnp
from jax import lax
from jax.experimental import pallas as pl
from jax.experimental.pallas import tpu as pltpu
from jax.experimental.pallas import tpu_sc as plsc

_N = 10000
_C = 128
_K = 16
_R = 2
_CO = _C * _R
_NP = 10240
_BLK = 128
_NBLK = _NP // _BLK

_NC = 2
_NS = 16
_NW = _NC * _NS
_NODES_PW = _NP // _NW
_CHUNK = 8
_NCHUNK = _NODES_PW // _CHUNK


def _proj_body(x_ref, wd_ref, w2_ref, bp_ref, a_ref, b_ref, xa_ref, xt_ref):
    x = x_ref[...]
    a_ref[...] = (
        jnp.dot(x, wd_ref[...], preferred_element_type=jnp.float32) + bp_ref[...]
    ).astype(jnp.bfloat16)
    b_ref[...] = jnp.dot(
        x, w2_ref[...], preferred_element_type=jnp.float32
    ).astype(jnp.bfloat16)
    sq = jnp.sum(x * x, axis=1, keepdims=True)
    xa_ref[...] = jnp.concatenate(
        [x, sq, jnp.ones((_BLK, 1), jnp.float32)], axis=1)
    xt_ref[...] = -2.0 * x.T


def _proj(xp, wd, w2, bp):
    return pl.pallas_call(
        _proj_body,
        grid=(_NBLK,),
        in_specs=[
            pl.BlockSpec((_BLK, _C), lambda i: (i, 0)),
            pl.BlockSpec((_C, _CO), lambda i: (0, 0)),
            pl.BlockSpec((_C, _CO), lambda i: (0, 0)),
            pl.BlockSpec((1, _CO), lambda i: (0, 0)),
        ],
        out_specs=[
            pl.BlockSpec((_BLK, _CO), lambda i: (i, 0)),
            pl.BlockSpec((_BLK, _CO), lambda i: (i, 0)),
            pl.BlockSpec((_BLK, _C + 2), lambda i: (i, 0)),
            pl.BlockSpec((_C, _BLK), lambda i: (0, i)),
        ],
        out_shape=[
            jax.ShapeDtypeStruct((_NP, _CO), jnp.bfloat16),
            jax.ShapeDtypeStruct((_NP, _CO), jnp.bfloat16),
            jax.ShapeDtypeStruct((_NP, _C + 2), jnp.float32),
            jax.ShapeDtypeStruct((_C, _NP), jnp.float32),
        ],
    )(xp, wd, w2, bp)


_G = 160
_S = _NP // _G


def _topk_body(xa_ref, xt_ref, nb_ref):
    xa = xa_ref[...]
    xts = xt_ref[...]
    sqb = 0.25 * jnp.sum(xts * xts, axis=0, keepdims=True)
    rhs = jnp.concatenate(
        [xts, jnp.ones((1, _BLK), jnp.float32), sqb], axis=0)
    d = jnp.dot(xa, rhs, preferred_element_type=jnp.float32)
    inf = jnp.float32(jnp.inf)

    d3 = d.reshape(_G, _S, _BLK)
    sio = lax.broadcasted_iota(jnp.int32, (_G, _S, _BLK), 1).astype(jnp.float32)
    sbig = jnp.float32(_S)

    def level(dd):
        g = jnp.min(dd, axis=1)
        eq = dd == g[:, None, :]
        a = jnp.min(jnp.where(eq, sio, sbig), axis=1)
        dd = jnp.where(eq, inf, dd)
        return g, a, dd

    g1, a1, d3 = level(d3)
    g2, a2, d3 = level(d3)
    g3, a3, _ = level(d3)

    gio = lax.broadcasted_iota(jnp.int32, (_G, _BLK), 0).astype(jnp.float32)
    kio = lax.broadcasted_iota(jnp.int32, (_K, _BLK), 0)
    nb = jnp.zeros((_K, _BLK), jnp.int32)
    gbig = jnp.float32(_G)
    for k in range(_K + 1):
        m = jnp.min(g1, axis=0, keepdims=True)
        candg = jnp.where(g1 == m, gio, gbig)
        gsel = jnp.min(candg, axis=0, keepdims=True)
        gm = gio == gsel
        amin = jnp.min(jnp.where(gm, a1, sbig), axis=0, keepdims=True)
        if k > 0:
            col = (gsel * _S + amin).astype(jnp.int32)
            nb = jnp.where(kio == k - 1, col, nb)
        g1 = jnp.where(gm, g2, g1)
        a1 = jnp.where(gm, a2, a1)
        g2 = jnp.where(gm, g3, g2)
        a2 = jnp.where(gm, a3, a2)
        g3 = jnp.where(gm, inf, g3)
    nb_ref[...] = nb


def _topk(xaug, xt):
    return pl.pallas_call(
        _topk_body,
        grid=(_NBLK,),
        in_specs=[
            pl.BlockSpec((_NP, _C + 2), lambda i: (0, 0)),
            pl.BlockSpec((_C, _BLK), lambda i: (0, i)),
        ],
        out_specs=pl.BlockSpec((_K, _BLK), lambda i: (0, i)),
        out_shape=jax.ShapeDtypeStruct((_K, _NP), jnp.int32),
    )(xaug, xt)


def _edge_body(nbf_hbm, bmat_hbm, amat_hbm, out_hbm,
               i0, i1, g0, g1, a0, a1, o0, o1,
               gs0, gs1, as0, as1, os0, os1):
    cid = lax.axis_index("c")
    sid = lax.axis_index("s")
    wid = sid * _NC + cid
    node0 = wid * _NODES_PW
    ibufs = (i0, i1)
    gbufs = (g0, g1)
    abufs = (a0, a1)
    obufs = (o0, o1)
    gsems = (gs0, gs1)
    asems = (as0, as1)
    osems = (os0, os1)

    def prefetch(c, b):
        pltpu.sync_copy(nbf_hbm.at[pl.ds(node0 * _K + c * (_CHUNK * _K),
                                         _CHUNK * _K)], ibufs[b])
        pltpu.async_copy(bmat_hbm.at[ibufs[b]], gbufs[b], gsems[b])
        pltpu.async_copy(amat_hbm.at[pl.ds(node0 + c * _CHUNK, _CHUNK)],
                         abufs[b], asems[b])

    for b in range(2):
        prefetch(b, b)

    def step(t, _):
        for b in range(2):
            c = 2 * t + b
            ib, gb, ab, ob = ibufs[b], gbufs[b], abufs[b], obufs[b]
            pltpu.make_async_copy(bmat_hbm.at[ib], gb, gsems[b]).wait()
            pltpu.make_async_copy(
                amat_hbm.at[pl.ds(node0, _CHUNK)], ab, asems[b]).wait()
            nbase = node0 + c * _CHUNK

            @pl.when(t >= 1)
            def _():
                pltpu.make_async_copy(
                    ob, out_hbm.at[pl.ds(node0, _CHUNK)], osems[b]).wait()

            def comp(cg, _):
                col = cg * 16
                bf = jnp.bfloat16
                for n in range(_CHUNK):
                    acc = plsc.bitcast(gb[n * _K, pl.ds(col, 16)], bf)
                    for j in range(1, _K):
                        acc = jnp.maximum(
                            acc, plsc.bitcast(gb[n * _K + j, pl.ds(col, 16)], bf))
                    av = plsc.bitcast(ab[n, pl.ds(col, 16)], bf)
                    ob[n, pl.ds(col, 16)] = plsc.bitcast(av + acc, jnp.int32)
                return 0

            lax.fori_loop(0, 8, comp, 0)
            pltpu.async_copy(ob, out_hbm.at[pl.ds(nbase, _CHUNK)], osems[b])

            @pl.when(c + 2 < _NCHUNK)
            def _():
                prefetch(c + 2, b)

        return 0

    lax.fori_loop(0, _NCHUNK // 2, step, 0)
    for b in range(2):
        pltpu.make_async_copy(
            obufs[b], out_hbm.at[pl.ds(node0, _CHUNK)], osems[b]).wait()


@functools.cache
def _make_edge():
  return functools.partial(
    pl.kernel,
    out_type=jax.ShapeDtypeStruct((_NP, _C), jnp.int32),
    mesh=plsc.VectorSubcoreMesh(core_axis_name="c", subcore_axis_name="s",
                                num_cores=_NC, num_subcores=_NS),
    compiler_params=pltpu.CompilerParams(needs_layout_passes=False),
    scratch_types=[
        pltpu.VMEM((_CHUNK * _K,), jnp.int32),
        pltpu.VMEM((_CHUNK * _K,), jnp.int32),
        pltpu.VMEM((_CHUNK * _K, _C), jnp.int32),
        pltpu.VMEM((_CHUNK * _K, _C), jnp.int32),
        pltpu.VMEM((_CHUNK, _C), jnp.int32),
        pltpu.VMEM((_CHUNK, _C), jnp.int32),
        pltpu.VMEM((_CHUNK, _C), jnp.int32),
        pltpu.VMEM((_CHUNK, _C), jnp.int32),
        pltpu.SemaphoreType.DMA,
        pltpu.SemaphoreType.DMA,
        pltpu.SemaphoreType.DMA,
        pltpu.SemaphoreType.DMA,
        pltpu.SemaphoreType.DMA,
        pltpu.SemaphoreType.DMA,
    ],
  )(_edge_body)


@jax.jit
def kernel(x, W, b):
    perm = jnp.concatenate([jnp.arange(0, _CO, 2), jnp.arange(1, _CO, 2)])
    Wp = W[:, perm]
    bp = b[perm].reshape(1, _CO)
    wd = Wp[:_C] - Wp[_C:]
    w2 = Wp[_C:]

    xp = jnp.pad(x, ((0, _NP - _N), (0, 0)), constant_values=1e4)

    a_mat, b_mat, xaug, xt = _proj(xp, wd, w2, bp)
    nb = _topk(xaug, xt)
    b_i = lax.bitcast_convert_type(b_mat.reshape(_NP, _C, 2), jnp.int32)
    a_i = lax.bitcast_convert_type(a_mat.reshape(_NP, _C, 2), jnp.int32)
    hp_i = _make_edge()(nb.T.reshape(-1), b_i, a_i)
    hp = lax.bitcast_convert_type(hp_i, jnp.bfloat16).reshape(_NP, _CO)
    return hp[:_N].astype(jnp.float32).reshape(_N * _R, _C)


# speedup vs baseline: 5.5275x; 5.5275x over previous
import functools

import jax
import jax.numpy as jnp
from jax import lax
from jax.experimental import pallas as pl
from jax.experimental.pallas import tpu as pltpu
from jax.experimental.pallas import tpu_sc as plsc

_N = 10000
_C = 128
_K = 16
_R = 2
_CO = _C * _R
_NP = 10240
_BLK = 128
_NBLK = _NP // _BLK

_NC = 2
_NS = 16
_NW = _NC * _NS
_NODES_PW = _NP // _NW
_CHUNK = 8
_NCHUNK = _NODES_PW // _CHUNK


def _proj_body(x_ref, wd_ref, w2_ref, bp_ref, a_ref, b_ref, xa_ref, xt_ref):
    x = x_ref[...]
    a_ref[...] = (
        jnp.dot(x, wd_ref[...], preferred_element_type=jnp.float32) + bp_ref[...]
    )
    b_ref[...] = jnp.dot(x, w2_ref[...], preferred_element_type=jnp.float32)
    sq = jnp.sum(x * x, axis=1, keepdims=True)
    xa_ref[...] = jnp.concatenate(
        [x, sq, jnp.ones((_BLK, 1), jnp.float32)], axis=1)
    xt_ref[...] = -2.0 * x.T


def _proj(xp, wd, w2, bp):
    return pl.pallas_call(
        _proj_body,
        grid=(_NBLK,),
        in_specs=[
            pl.BlockSpec((_BLK, _C), lambda i: (i, 0)),
            pl.BlockSpec((_C, _CO), lambda i: (0, 0)),
            pl.BlockSpec((_C, _CO), lambda i: (0, 0)),
            pl.BlockSpec((1, _CO), lambda i: (0, 0)),
        ],
        out_specs=[
            pl.BlockSpec((_BLK, _CO), lambda i: (i, 0)),
            pl.BlockSpec((_BLK, _CO), lambda i: (i, 0)),
            pl.BlockSpec((_BLK, _C + 2), lambda i: (i, 0)),
            pl.BlockSpec((_C, _BLK), lambda i: (0, i)),
        ],
        out_shape=[
            jax.ShapeDtypeStruct((_NP, _CO), jnp.float32),
            jax.ShapeDtypeStruct((_NP, _CO), jnp.float32),
            jax.ShapeDtypeStruct((_NP, _C + 2), jnp.float32),
            jax.ShapeDtypeStruct((_C, _NP), jnp.float32),
        ],
    )(xp, wd, w2, bp)


_G = 160
_S = _NP // _G


def _topk_body(xa_ref, xt_ref, nb_ref):
    xa = xa_ref[...]
    xts = xt_ref[...]
    sqb = 0.25 * jnp.sum(xts * xts, axis=0, keepdims=True)
    rhs = jnp.concatenate(
        [xts, jnp.ones((1, _BLK), jnp.float32), sqb], axis=0)
    d = jnp.dot(xa, rhs, preferred_element_type=jnp.float32)
    inf = jnp.float32(jnp.inf)

    d3 = d.reshape(_G, _S, _BLK)
    sio = lax.broadcasted_iota(jnp.int32, (_G, _S, _BLK), 1).astype(jnp.float32)
    sbig = jnp.float32(_S)

    def level(dd):
        g = jnp.min(dd, axis=1)
        eq = dd == g[:, None, :]
        a = jnp.min(jnp.where(eq, sio, sbig), axis=1)
        dd = jnp.where(eq, inf, dd)
        return g, a, dd

    g1, a1, d3 = level(d3)
    g2, a2, d3 = level(d3)
    g3, a3, _ = level(d3)

    gio = lax.broadcasted_iota(jnp.int32, (_G, _BLK), 0).astype(jnp.float32)
    kio = lax.broadcasted_iota(jnp.int32, (_K, _BLK), 0)
    nb = jnp.zeros((_K, _BLK), jnp.int32)
    gbig = jnp.float32(_G)
    for k in range(_K + 1):
        m = jnp.min(g1, axis=0, keepdims=True)
        candg = jnp.where(g1 == m, gio, gbig)
        gsel = jnp.min(candg, axis=0, keepdims=True)
        gm = gio == gsel
        amin = jnp.min(jnp.where(gm, a1, sbig), axis=0, keepdims=True)
        if k > 0:
            col = (gsel * _S + amin).astype(jnp.int32)
            nb = jnp.where(kio == k - 1, col, nb)
        g1 = jnp.where(gm, g2, g1)
        a1 = jnp.where(gm, a2, a1)
        g2 = jnp.where(gm, g3, g2)
        a2 = jnp.where(gm, a3, a2)
        g3 = jnp.where(gm, inf, g3)
    nb_ref[...] = nb


def _topk(xaug, xt):
    return pl.pallas_call(
        _topk_body,
        grid=(_NBLK,),
        in_specs=[
            pl.BlockSpec((_NP, _C + 2), lambda i: (0, 0)),
            pl.BlockSpec((_C, _BLK), lambda i: (0, i)),
        ],
        out_specs=pl.BlockSpec((_K, _BLK), lambda i: (0, i)),
        out_shape=jax.ShapeDtypeStruct((_K, _NP), jnp.int32),
    )(xaug, xt)


def _edge_body(nbf_hbm, bmat_hbm, amat_hbm, out_hbm,
               i0, i1, g0, g1, a0, a1, o0, o1,
               gs0, gs1, as0, as1, os0, os1):
    cid = lax.axis_index("c")
    sid = lax.axis_index("s")
    wid = sid * _NC + cid
    node0 = wid * _NODES_PW
    ibufs = (i0, i1)
    gbufs = (g0, g1)
    abufs = (a0, a1)
    obufs = (o0, o1)
    gsems = (gs0, gs1)
    asems = (as0, as1)
    osems = (os0, os1)

    def prefetch(c, b):
        pltpu.sync_copy(nbf_hbm.at[pl.ds(node0 * _K + c * (_CHUNK * _K),
                                         _CHUNK * _K)], ibufs[b])
        pltpu.async_copy(bmat_hbm.at[ibufs[b]], gbufs[b], gsems[b])
        pltpu.async_copy(amat_hbm.at[pl.ds(node0 + c * _CHUNK, _CHUNK)],
                         abufs[b], asems[b])

    for b in range(2):
        prefetch(b, b)

    def step(t, _):
        for b in range(2):
            c = 2 * t + b
            ib, gb, ab, ob = ibufs[b], gbufs[b], abufs[b], obufs[b]
            pltpu.make_async_copy(bmat_hbm.at[ib], gb, gsems[b]).wait()
            pltpu.make_async_copy(
                amat_hbm.at[pl.ds(node0, _CHUNK)], ab, asems[b]).wait()
            nbase = node0 + c * _CHUNK

            @pl.when(t >= 1)
            def _():
                pltpu.make_async_copy(
                    ob, out_hbm.at[pl.ds(node0, _CHUNK)], osems[b]).wait()

            def comp(cg, _):
                col = cg * 16
                for n in range(_CHUNK):
                    acc = gb[n * _K, pl.ds(col, 16)]
                    for j in range(1, _K):
                        acc = jnp.maximum(acc, gb[n * _K + j, pl.ds(col, 16)])
                    ob[n, pl.ds(col, 16)] = ab[n, pl.ds(col, 16)] + acc
                return 0

            lax.fori_loop(0, 16, comp, 0)
            pltpu.async_copy(ob, out_hbm.at[pl.ds(nbase, _CHUNK)], osems[b])

            @pl.when(c + 2 < _NCHUNK)
            def _():
                prefetch(c + 2, b)

        return 0

    lax.fori_loop(0, _NCHUNK // 2, step, 0)
    for b in range(2):
        pltpu.make_async_copy(
            obufs[b], out_hbm.at[pl.ds(node0, _CHUNK)], osems[b]).wait()


@functools.cache
def _make_edge():
  return functools.partial(
    pl.kernel,
    out_type=jax.ShapeDtypeStruct((_NP, _CO), jnp.float32),
    mesh=plsc.VectorSubcoreMesh(core_axis_name="c", subcore_axis_name="s",
                                num_cores=_NC, num_subcores=_NS),
    scratch_types=[
        pltpu.VMEM((_CHUNK * _K,), jnp.int32),
        pltpu.VMEM((_CHUNK * _K,), jnp.int32),
        pltpu.VMEM((_CHUNK * _K, _CO), jnp.float32),
        pltpu.VMEM((_CHUNK * _K, _CO), jnp.float32),
        pltpu.VMEM((_CHUNK, _CO), jnp.float32),
        pltpu.VMEM((_CHUNK, _CO), jnp.float32),
        pltpu.VMEM((_CHUNK, _CO), jnp.float32),
        pltpu.VMEM((_CHUNK, _CO), jnp.float32),
        pltpu.SemaphoreType.DMA,
        pltpu.SemaphoreType.DMA,
        pltpu.SemaphoreType.DMA,
        pltpu.SemaphoreType.DMA,
        pltpu.SemaphoreType.DMA,
        pltpu.SemaphoreType.DMA,
    ],
  )(_edge_body)


@jax.jit
def kernel(x, W, b):
    perm = jnp.concatenate([jnp.arange(0, _CO, 2), jnp.arange(1, _CO, 2)])
    Wp = W[:, perm]
    bp = b[perm].reshape(1, _CO)
    wd = Wp[:_C] - Wp[_C:]
    w2 = Wp[_C:]

    xp = jnp.pad(x, ((0, _NP - _N), (0, 0)), constant_values=1e4)

    a_mat, b_mat, xaug, xt = _proj(xp, wd, w2, bp)
    nb = _topk(xaug, xt)
    hp = _make_edge()(nb.T.reshape(-1), b_mat, a_mat)
    return hp[:_N].reshape(_N * _R, _C)


# speedup vs baseline: 5.6453x; 1.0213x over previous
import functools

import jax
import jax.numpy as jnp
from jax import lax
from jax.experimental import pallas as pl
from jax.experimental.pallas import tpu as pltpu
from jax.experimental.pallas import tpu_sc as plsc

_N = 10000
_C = 128
_K = 16
_R = 2
_CO = _C * _R
_NP = 10240
_BLK = 128
_NBLK = _NP // _BLK

_NC = 2
_NS = 16
_NW = _NC * _NS
_NODES_PW = _NP // _NW
_CHUNK = 8
_NCHUNK = _NODES_PW // _CHUNK


def _proj_body(x_ref, wd_ref, w2_ref, bp_ref, a_ref, b_ref, sq_ref, xt_ref):
    x = x_ref[...]
    a_ref[...] = (
        jnp.dot(x, wd_ref[...], preferred_element_type=jnp.float32) + bp_ref[...]
    )
    b_ref[...] = jnp.dot(x, w2_ref[...], preferred_element_type=jnp.float32)
    sq_ref[...] = jnp.sum(x * x, axis=1, keepdims=True)
    xt_ref[...] = -2.0 * x.T


def _proj(xp, wd, w2, bp):
    return pl.pallas_call(
        _proj_body,
        grid=(_NBLK,),
        in_specs=[
            pl.BlockSpec((_BLK, _C), lambda i: (i, 0)),
            pl.BlockSpec((_C, _CO), lambda i: (0, 0)),
            pl.BlockSpec((_C, _CO), lambda i: (0, 0)),
            pl.BlockSpec((1, _CO), lambda i: (0, 0)),
        ],
        out_specs=[
            pl.BlockSpec((_BLK, _CO), lambda i: (i, 0)),
            pl.BlockSpec((_BLK, _CO), lambda i: (i, 0)),
            pl.BlockSpec((_BLK, 1), lambda i: (i, 0)),
            pl.BlockSpec((_C, _BLK), lambda i: (0, i)),
        ],
        out_shape=[
            jax.ShapeDtypeStruct((_NP, _CO), jnp.float32),
            jax.ShapeDtypeStruct((_NP, _CO), jnp.float32),
            jax.ShapeDtypeStruct((_NP, 1), jnp.float32),
            jax.ShapeDtypeStruct((_C, _NP), jnp.float32),
        ],
    )(xp, wd, w2, bp)


_G = 160
_S = _NP // _G


def _topk_body(x_ref, xt_ref, sq_ref, nb_ref):
    x = x_ref[...]
    xts = xt_ref[...]
    sqc = sq_ref[...]
    sqb = 0.25 * jnp.sum(xts * xts, axis=0, keepdims=True)
    d = sqc + sqb + jnp.dot(x, xts, preferred_element_type=jnp.float32)
    inf = jnp.float32(jnp.inf)

    d3 = d.reshape(_G, _S, _BLK)
    sio = lax.broadcasted_iota(jnp.int32, (_G, _S, _BLK), 1).astype(jnp.float32)
    sbig = jnp.float32(_S)

    def level(dd):
        g = jnp.min(dd, axis=1)
        eq = dd == g[:, None, :]
        a = jnp.min(jnp.where(eq, sio, sbig), axis=1)
        dd = jnp.where(eq, inf, dd)
        return g, a, dd

    g1, a1, d3 = level(d3)
    g2, a2, d3 = level(d3)
    g3, a3, _ = level(d3)

    gio = lax.broadcasted_iota(jnp.int32, (_G, _BLK), 0).astype(jnp.float32)
    kio = lax.broadcasted_iota(jnp.int32, (_K, _BLK), 0)
    nb = jnp.zeros((_K, _BLK), jnp.int32)
    gbig = jnp.float32(_G)
    for k in range(_K + 1):
        m = jnp.min(g1, axis=0, keepdims=True)
        candg = jnp.where(g1 == m, gio, gbig)
        gsel = jnp.min(candg, axis=0, keepdims=True)
        gm = gio == gsel
        amin = jnp.min(jnp.where(gm, a1, sbig), axis=0, keepdims=True)
        if k > 0:
            col = (gsel * _S + amin).astype(jnp.int32)
            nb = jnp.where(kio == k - 1, col, nb)
        g1 = jnp.where(gm, g2, g1)
        a1 = jnp.where(gm, a2, a1)
        g2 = jnp.where(gm, g3, g2)
        a2 = jnp.where(gm, a3, a2)
        g3 = jnp.where(gm, inf, g3)
    nb_ref[...] = nb


def _topk(xp, xt, sqc):
    return pl.pallas_call(
        _topk_body,
        grid=(_NBLK,),
        in_specs=[
            pl.BlockSpec((_NP, _C), lambda i: (0, 0)),
            pl.BlockSpec((_C, _BLK), lambda i: (0, i)),
            pl.BlockSpec((_NP, 1), lambda i: (0, 0)),
        ],
        out_specs=pl.BlockSpec((_K, _BLK), lambda i: (0, i)),
        out_shape=jax.ShapeDtypeStruct((_K, _NP), jnp.int32),
    )(xp, xt, sqc)


def _edge_body(nbf_hbm, bmat_hbm, amat_hbm, out_hbm,
               i0, i1, g0, g1, a0, a1, o0, o1,
               gs0, gs1, as0, as1, os0, os1):
    cid = lax.axis_index("c")
    sid = lax.axis_index("s")
    wid = sid * _NC + cid
    node0 = wid * _NODES_PW
    ibufs = (i0, i1)
    gbufs = (g0, g1)
    abufs = (a0, a1)
    obufs = (o0, o1)
    gsems = (gs0, gs1)
    asems = (as0, as1)
    osems = (os0, os1)

    def prefetch(c, b):
        pltpu.sync_copy(nbf_hbm.at[pl.ds(node0 * _K + c * (_CHUNK * _K),
                                         _CHUNK * _K)], ibufs[b])
        pltpu.async_copy(bmat_hbm.at[ibufs[b]], gbufs[b], gsems[b])
        pltpu.async_copy(amat_hbm.at[pl.ds(node0 + c * _CHUNK, _CHUNK)],
                         abufs[b], asems[b])

    for b in range(2):
        prefetch(b, b)

    def step(t, _):
        for b in range(2):
            c = 2 * t + b
            ib, gb, ab, ob = ibufs[b], gbufs[b], abufs[b], obufs[b]
            pltpu.make_async_copy(bmat_hbm.at[ib], gb, gsems[b]).wait()
            pltpu.make_async_copy(
                amat_hbm.at[pl.ds(node0, _CHUNK)], ab, asems[b]).wait()
            nbase = node0 + c * _CHUNK

            @pl.when(t >= 1)
            def _():
                pltpu.make_async_copy(
                    ob, out_hbm.at[pl.ds(node0, _CHUNK)], osems[b]).wait()

            def comp(cg, _):
                col = cg * 16
                for n in range(_CHUNK):
                    acc = gb[n * _K, pl.ds(col, 16)]
                    for j in range(1, _K):
                        acc = jnp.maximum(acc, gb[n * _K + j, pl.ds(col, 16)])
                    ob[n, pl.ds(col, 16)] = ab[n, pl.ds(col, 16)] + acc
                return 0

            lax.fori_loop(0, 16, comp, 0)
            pltpu.async_copy(ob, out_hbm.at[pl.ds(nbase, _CHUNK)], osems[b])

            @pl.when(c + 2 < _NCHUNK)
            def _():
                prefetch(c + 2, b)

        return 0

    lax.fori_loop(0, _NCHUNK // 2, step, 0)
    for b in range(2):
        pltpu.make_async_copy(
            obufs[b], out_hbm.at[pl.ds(node0, _CHUNK)], osems[b]).wait()


@functools.cache
def _make_edge():
  return functools.partial(
    pl.kernel,
    out_type=jax.ShapeDtypeStruct((_NP, _CO), jnp.float32),
    mesh=plsc.VectorSubcoreMesh(core_axis_name="c", subcore_axis_name="s",
                                num_cores=_NC, num_subcores=_NS),
    scratch_types=[
        pltpu.VMEM((_CHUNK * _K,), jnp.int32),
        pltpu.VMEM((_CHUNK * _K,), jnp.int32),
        pltpu.VMEM((_CHUNK * _K, _CO), jnp.float32),
        pltpu.VMEM((_CHUNK * _K, _CO), jnp.float32),
        pltpu.VMEM((_CHUNK, _CO), jnp.float32),
        pltpu.VMEM((_CHUNK, _CO), jnp.float32),
        pltpu.VMEM((_CHUNK, _CO), jnp.float32),
        pltpu.VMEM((_CHUNK, _CO), jnp.float32),
        pltpu.SemaphoreType.DMA,
        pltpu.SemaphoreType.DMA,
        pltpu.SemaphoreType.DMA,
        pltpu.SemaphoreType.DMA,
        pltpu.SemaphoreType.DMA,
        pltpu.SemaphoreType.DMA,
    ],
  )(_edge_body)


@jax.jit
def kernel(x, W, b):
    perm = jnp.concatenate([jnp.arange(0, _CO, 2), jnp.arange(1, _CO, 2)])
    Wp = W[:, perm]
    bp = b[perm].reshape(1, _CO)
    wd = Wp[:_C] - Wp[_C:]
    w2 = Wp[_C:]

    xp = jnp.pad(x, ((0, _NP - _N), (0, 0)), constant_values=1e4)

    a_mat, b_mat, sqc, xt = _proj(xp, wd, w2, bp)
    nb = _topk(xp, xt, sqc)
    hp = _make_edge()(nb.T.reshape(-1), b_mat, a_mat)
    return hp[:_N].reshape(_N * _R, _C)
